# Initial kernel scaffold; baseline (speedup 1.0000x reference)
import jax
import jax.numpy as jnp
from jax.experimental import pallas as pl


def kernel(cat_seq, cont_seq, ET, EC, AR, JT, SP, PT, W_cont, b_cont, W_out, b_out, gamma, beta):
    raise NotImplementedError("write your pallas kernel here")


# speedup vs baseline: 4.5459x; 4.5459x over previous
import functools

import jax
import jax.numpy as jnp
from jax.experimental import pallas as pl
from jax.experimental.pallas import tpu as pltpu
from jax.experimental.pallas import tpu_sc as plsc

_LIST = 128
_R = 2
_BT = 2048


def _sc_gather(tables, idx6, n):
    dims = [int(t.shape[1]) for t in tables]
    C = n // _LIST
    mesh = plsc.VectorSubcoreMesh(core_axis_name="c", subcore_axis_name="s")
    out_type = [jax.ShapeDtypeStruct((n, d), jnp.float32) for d in dims]

    @functools.partial(
        pl.kernel,
        out_type=out_type,
        mesh=mesh,
        scratch_types=[pltpu.SemaphoreType.DMA] * 6,
        compiler_params=pltpu.CompilerParams(use_tc_tiling_on_sc=False),
    )
    def gather_kernel(t0, t1, t2, t3, t4, t5,
                      i0, i1, i2, i3, i4, i5,
                      o0, o1, o2, o3, o4, o5,
                      s0, s1, s2, s3, s4, s5):
        tbls = (t0, t1, t2, t3, t4, t5)
        sems = (s0, s1, s2, s3, s4, s5)

        def body(*refs):
            ivs = refs[:6]
            ovs = refs[6:12]
            cps = []
            for r in range(_R):
                for k in range(6):
                    cps.append(pltpu.make_async_copy(
                        tbls[k].at[ivs[k].at[r]],
                        ovs[k].at[pl.ds(r * _LIST, _LIST)],
                        sems[k]))
            for cp in cps:
                cp.start()
            for cp in cps:
                cp.wait()

        pltpu.emit_pipeline(
            body,
            grid=(C // _R,),
            in_specs=[pl.BlockSpec((_R, _LIST), lambda i: (i, 0))
                      for _ in range(6)],
            out_specs=[pl.BlockSpec((_R * _LIST, d), lambda i: (i, 0))
                       for d in dims],
            core_axis_name=("c", "s"),
            dimension_semantics=(pltpu.PARALLEL,),
        )(i0, i1, i2, i3, i4, i5, o0, o1, o2, o3, o4, o5)

    return gather_kernel(*tables, *idx6)


def _tc_body(e0, e1, e2, e3, e4, e5, cont, wc, bc, wo, bo, g, bt, out):
    c = jnp.maximum(
        jax.lax.dot_general(
            cont[...].astype(jnp.bfloat16), wc[...],
            (((1,), (0,)), ((), ())),
            preferred_element_type=jnp.float32) + bc[...],
        0.0)
    x = jnp.concatenate(
        [e0[...], e1[...], e2[...], e3[...], e4[...], e5[...], c], axis=1)
    h = jax.lax.dot_general(
        x.astype(jnp.bfloat16), wo[...],
        (((1,), (0,)), ((), ())),
        preferred_element_type=jnp.float32) + bo[...]
    m = jnp.mean(h, axis=1, keepdims=True)
    v = jnp.mean((h - m) ** 2, axis=1, keepdims=True)
    hn = (h - m) * jax.lax.rsqrt(v + 1e-5) * g[...] + bt[...]
    out[...] = jnp.maximum(hn, 0.0)


def _tc_dense(es, cont, W_cont, b_cont, W_out, b_out, gamma, beta):
    n = cont.shape[0]
    dims = [int(e.shape[1]) for e in es]

    def block(shape):
        return pl.BlockSpec(shape, lambda i: (i,) + (0,) * (len(shape) - 1))

    def fixed(shape):
        return pl.BlockSpec(shape, lambda i: (0,) * len(shape))

    return pl.pallas_call(
        _tc_body,
        grid=(n // _BT,),
        in_specs=[block((_BT, d)) for d in dims] + [
            block((_BT, 16)),
            fixed((16, 56)),
            fixed((1, 56)),
            fixed((176, 128)),
            fixed((1, 128)),
            fixed((1, 128)),
            fixed((1, 128)),
        ],
        out_specs=block((_BT, 128)),
        out_shape=jax.ShapeDtypeStruct((n, 128), jnp.float32),
    )(*es, cont, W_cont, b_cont, W_out, b_out, gamma, beta)


def kernel(cat_seq, cont_seq, ET, EC, AR, JT, SP, PT,
           W_cont, b_cont, W_out, b_out, gamma, beta):
    Bb, Ll, _ = cat_seq.shape
    n = Bb * Ll
    idx = cat_seq.astype(jnp.int32).reshape(n, 6).T
    idx6 = [idx[k].reshape(n // _LIST, _LIST) for k in range(6)]
    es = _sc_gather([ET, EC, AR, JT, SP, PT], idx6, n)
    out = _tc_dense(
        list(es),
        cont_seq.reshape(n, 16),
        W_cont.astype(jnp.bfloat16),
        b_cont.reshape(1, 56),
        W_out.astype(jnp.bfloat16),
        b_out.reshape(1, 128),
        gamma.reshape(1, 128),
        beta.reshape(1, 128),
    )
    return out.reshape(Bb, Ll, 128)


# speedup vs baseline: 9.0492x; 1.9906x over previous
import functools

import jax
import jax.numpy as jnp
from jax.experimental import pallas as pl
from jax.experimental.pallas import tpu as pltpu
from jax.experimental.pallas import tpu_sc as plsc

_LIST = 128
_R = 2
_BT = 2048

_FIELDS = ((0, 0, 32), (1, 32, 16), (2, 48, 16),
           (3, 64, 32), (4, 96, 16), (5, 112, 8))


_DIMS = (32, 16, 16, 32, 16, 8)
_W7 = _FIELDS + ((5, 120, 8),)


def _sc_gather(tables, cat, n):
    mesh = plsc.VectorSubcoreMesh(core_axis_name="c", subcore_axis_name="s")
    out_type = jax.ShapeDtypeStruct((n, 128), jnp.float32)
    NW = 32
    LPT = n // (_LIST * NW)

    @functools.partial(
        pl.kernel,
        out_type=out_type,
        mesh=mesh,
        scratch_types=(
            [pltpu.VMEM((2, _LIST, 6), jnp.int32),
             pltpu.VMEM((2, 6, _LIST), jnp.int32)] +
            [pltpu.VMEM((2, _LIST, d), jnp.float32) for d in _DIMS] +
            [pltpu.SemaphoreType.DMA] * 6
        ),
        compiler_params=pltpu.CompilerParams(
            use_tc_tiling_on_sc=False, needs_layout_passes=False),
    )
    def gather_kernel(t0, t1, t2, t3, t4, t5, cat_hbm, o_hbm, *scratch):
        tbls = (t0, t1, t2, t3, t4, t5)
        cat_buf, ilist = scratch[0], scratch[1]
        comp = scratch[2:8]
        csems = scratch[8:10]
        gsems = scratch[10:12]
        wsems = scratch[12:14]

        wid = jax.lax.axis_index("s") * 2 + jax.lax.axis_index("c")
        base = wid * LPT

        def cat_copy(g, b):
            return pltpu.make_async_copy(
                cat_hbm.at[pl.ds((base + g) * _LIST, _LIST)],
                cat_buf.at[b], csems[b])

        def write_copies(row0, b):
            return [pltpu.make_async_copy(
                        comp[k].at[b],
                        o_hbm.at[pl.ds(row0, _LIST), pl.ds(off, d)],
                        wsems[b])
                    for (k, off, d) in _W7]

        cat_copy(0, 0).start()
        cat_copy(1, 1).start()

        @pl.loop(0, LPT, step=2)
        def _(g):
            for b in range(2):
                gg = g + b
                cat_copy(gg, b).wait()
                lane = jax.lax.iota(jnp.int32, 16)
                for k in range(6):
                    col = jnp.full((16,), k, jnp.int32)
                    for j in range(_LIST // 16):
                        ilist[b, k, pl.ds(j * 16, 16)] = plsc.load_gather(
                            cat_buf.at[b], [lane + j * 16, col])

                @pl.when(gg >= 2)
                def _():
                    for w in write_copies((base + gg - 2) * _LIST, b):
                        w.wait()

                gathers = [pltpu.make_async_copy(
                               tbls[k].at[ilist.at[b, k]],
                               comp[k].at[b], gsems[b])
                           for k in range(6)]
                for cp in gathers:
                    cp.start()

                @pl.when(gg + 2 < LPT)
                def _():
                    cat_copy(gg + 2, b).start()

                for cp in gathers:
                    cp.wait()
                for w in write_copies((base + gg) * _LIST, b):
                    w.start()

        for b in range(2):
            for w in write_copies((base + LPT - 2 + b) * _LIST, b):
                w.wait()

    return gather_kernel(*tables, cat)


def _tc_body(e, cont, wc, bc, w1, w2, bo, g, bt, out):
    c = jnp.maximum(
        jax.lax.dot_general(
            cont[...].astype(jnp.bfloat16), wc[...],
            (((1,), (0,)), ((), ())),
            preferred_element_type=jnp.float32) + bc[...],
        0.0)
    h = jax.lax.dot_general(
        e[...].astype(jnp.bfloat16), w1[...],
        (((1,), (0,)), ((), ())),
        preferred_element_type=jnp.float32)
    h = h + jax.lax.dot_general(
        c.astype(jnp.bfloat16), w2[...],
        (((1,), (0,)), ((), ())),
        preferred_element_type=jnp.float32) + bo[...]
    m = jnp.mean(h, axis=1, keepdims=True)
    v = jnp.mean((h - m) ** 2, axis=1, keepdims=True)
    hn = (h - m) * jax.lax.rsqrt(v + 1e-5) * g[...] + bt[...]
    out[...] = jnp.maximum(hn, 0.0)


def _tc_dense(e, cont, W_cont, b_cont, W1, W2, b_out, gamma, beta):
    n = cont.shape[0]

    def block(shape):
        return pl.BlockSpec(shape, lambda i: (i,) + (0,) * (len(shape) - 1))

    def fixed(shape):
        return pl.BlockSpec(shape, lambda i: (0,) * len(shape))

    return pl.pallas_call(
        _tc_body,
        grid=(n // _BT,),
        in_specs=[
            block((_BT, 128)),
            block((_BT, 16)),
            fixed((16, 56)),
            fixed((1, 56)),
            fixed((128, 128)),
            fixed((56, 128)),
            fixed((1, 128)),
            fixed((1, 128)),
            fixed((1, 128)),
        ],
        out_specs=block((_BT, 128)),
        out_shape=jax.ShapeDtypeStruct((n, 128), jnp.float32),
    )(e, cont, W_cont, b_cont, W1, W2, b_out, gamma, beta)


def kernel(cat_seq, cont_seq, ET, EC, AR, JT, SP, PT,
           W_cont, b_cont, W_out, b_out, gamma, beta):
    Bb, Ll, _ = cat_seq.shape
    n = Bb * Ll
    cat = cat_seq.astype(jnp.int32).reshape(n, 6)
    e = _sc_gather([ET, EC, AR, JT, SP, PT], cat, n)
    W1 = jnp.concatenate(
        [W_out[:120], jnp.zeros((8, 128), W_out.dtype)], axis=0)
    out = _tc_dense(
        e,
        cont_seq.reshape(n, 16),
        W_cont.astype(jnp.bfloat16),
        b_cont.reshape(1, 56),
        W1.astype(jnp.bfloat16),
        W_out[120:176].astype(jnp.bfloat16),
        b_out.reshape(1, 128),
        gamma.reshape(1, 128),
        beta.reshape(1, 128),
    )
    return out.reshape(Bb, Ll, 128)
